# baseline (device time: 11203 ns/iter reference)
import jax
import jax.numpy as jnp
from jax import lax
from jax.experimental import pallas as pl
from jax.experimental.pallas import tpu as pltpu

N_DEV = 16


def kernel(x):
    m, n = x.shape
    dtype = jnp.float32

    def body(x_ref, out_ref, cnt_ref, send_buf, recv_bufs, send_sems,
             recv_sems):
        k = lax.axis_index("i")
        c = cnt_ref[0]
        cnt_ref[0] = c + 1
        p = jnp.bitwise_and(c, 1)

        barrier_sem = pltpu.get_barrier_semaphore()

        @pl.when(c == 0)
        def _():
            for o in range(1, N_DEV):
                pl.semaphore_signal(
                    barrier_sem, inc=1,
                    device_id=(lax.rem(k + o, N_DEV),),
                    device_id_type=pl.DeviceIdType.MESH,
                )
            pl.semaphore_wait(barrier_sem, N_DEV - 1)

        t = x_ref[:, :]
        size = m
        while size > 1:
            half = size // 2
            t = t[:half, :] * t[half:size, :]
            size = half
        send_buf[:, :] = t

        for bank in (0, 1):
            for o in range(N_DEV - 1, 0, -1):
                @pl.when(p == bank)
                def _(o=o, bank=bank):
                    snd = pltpu.make_async_remote_copy(
                        src_ref=send_buf,
                        dst_ref=recv_bufs.at[bank, o - 1],
                        send_sem=send_sems.at[o - 1],
                        recv_sem=recv_sems.at[bank, o - 1],
                        device_id=(lax.rem(k + o, N_DEV),),
                        device_id_type=pl.DeviceIdType.MESH,
                    )
                    snd.start()

        y = x_ref[:, :]
        d = 1
        while d < m:
            y = y * jnp.concatenate(
                [jnp.ones((d, n), dtype), y[: m - d, :]], axis=0
            )
            d *= 2

        for bank in (0, 1):
            for s in range(N_DEV - 1):
                @pl.when(p == bank)
                def _(s=s, bank=bank):
                    rcv = pltpu.make_async_remote_copy(
                        src_ref=send_buf,
                        dst_ref=recv_bufs.at[bank, s],
                        send_sem=send_sems.at[s],
                        recv_sem=recv_sems.at[bank, s],
                        device_id=(k,),
                        device_id_type=pl.DeviceIdType.MESH,
                    )
                    rcv.wait_recv()

        r = jnp.where(p == 0, recv_bufs[0, :, 0, :], recv_bufs[1, :, 0, :])
        row = lax.broadcasted_iota(jnp.int32, (N_DEV - 1, n), 0)
        w = jnp.where(row < k, r, jnp.ones_like(r))
        w = jnp.concatenate([w, jnp.ones((1, n), dtype)], axis=0)
        size = N_DEV
        while size > 1:
            half = size // 2
            w = w[:half, :] * w[half:size, :]
            size = half

        out_ref[:, :] = y * w

        for bank in (0, 1):
            for o in range(1, N_DEV):
                @pl.when(p == bank)
                def _(o=o, bank=bank):
                    snd = pltpu.make_async_remote_copy(
                        src_ref=send_buf,
                        dst_ref=recv_bufs.at[bank, o - 1],
                        send_sem=send_sems.at[o - 1],
                        recv_sem=recv_sems.at[bank, o - 1],
                        device_id=(lax.rem(k + o, N_DEV),),
                        device_id_type=pl.DeviceIdType.MESH,
                    )
                    snd.wait_send()

    return pl.pallas_call(
        body,
        out_shape=jax.ShapeDtypeStruct((m, n), dtype),
        in_specs=[pl.BlockSpec(memory_space=pltpu.VMEM)],
        out_specs=pl.BlockSpec(memory_space=pltpu.VMEM),
        scratch_shapes=[
            pltpu.SMEM((1,), jnp.int32),
            pltpu.VMEM((1, n), dtype),
            pltpu.VMEM((2, N_DEV - 1, 1, n), dtype),
            pltpu.SemaphoreType.DMA((N_DEV - 1,)),
            pltpu.SemaphoreType.DMA((2, N_DEV - 1)),
        ],
        compiler_params=pltpu.CompilerParams(collective_id=0),
    )(x)
